# baseline (speedup 1.0000x reference)
import functools

import jax
import jax.numpy as jnp
from jax import lax
from jax.experimental import pallas as pl
from jax.experimental.pallas import tpu as pltpu
from jax.experimental.pallas import tpu_sc as plsc

B, F, V, D = 16384, 66, 100000, 4
H1, H2 = 16, 8
RW = 8
NI = B * F
NC, NS = 2, 16
NW = NC * NS
IPG = 128

SB = 64
SPW = B // NW
NST = SPW // SB
GPS = SB // 16
IPS = SB * F
GA = IPS // IPG

OFF_W1 = 0
OFF_B1 = OFF_W1 + F * D * H1
OFF_W2 = OFF_B1 + H1 * 16
OFF_B2 = OFF_W2 + H2 * H1 * 16
OFF_W3 = OFF_B2 + H2 * 16
OFF_B3 = OFF_W3 + H2 * 16
WPK = OFF_B3 + 16

TB_CH = 2000
TB_NCH = V // TB_CH
TB_IT = -(-TB_NCH // NW)

_MESH = dict(core_axis_name="c", subcore_axis_name="s")
_CP = pltpu.CompilerParams(use_tc_tiling_on_sc=False, needs_layout_passes=False)


def _sc_build_tab(emb_wT, emb_b):
    @functools.partial(
        pl.kernel, mesh=plsc.VectorSubcoreMesh(**_MESH),
        compiler_params=_CP,
        out_type=jax.ShapeDtypeStruct((V, RW), jnp.float32),
        scratch_types=[
            pltpu.VMEM((D, TB_CH), jnp.float32),
            pltpu.VMEM((TB_CH,), jnp.float32),
            pltpu.VMEM((TB_CH, RW), jnp.float32),
        ],
    )
    def k(wT_hbm, b_hbm, tab_hbm, w_v, b_v, rows_v):
        wid = lax.axis_index("s") * NC + lax.axis_index("c")
        for it in range(TB_IT):
            c = wid + NW * it

            @pl.when(c < TB_NCH)
            def _():
                r0 = c * TB_CH
                pltpu.sync_copy(wT_hbm.at[:, pl.ds(r0, TB_CH)], w_v)
                pltpu.sync_copy(b_hbm.at[pl.ds(r0, TB_CH)], b_v)

                def body(g, carry):
                    rows = jnp.arange(16, dtype=jnp.int32) + g * 16
                    for d in range(D):
                        plsc.store_scatter(
                            rows_v,
                            [rows, jnp.full((16,), d, jnp.int32)],
                            w_v.at[d][pl.ds(g * 16, 16)])
                    plsc.store_scatter(
                        rows_v,
                        [rows, jnp.full((16,), D, jnp.int32)],
                        b_v[pl.ds(g * 16, 16)])
                    return carry

                lax.fori_loop(0, TB_CH // 16, body, 0)
                pltpu.sync_copy(rows_v, tab_hbm.at[pl.ds(r0, TB_CH)])

    return k(emb_wT, emb_b)


def _bcast(vec16, j):
    return lax.gather(
        vec16, jnp.full((16, 1), j, jnp.int32),
        lax.GatherDimensionNumbers(offset_dims=(), collapsed_slice_dims=(0,),
                                   start_index_map=(0,)),
        (1,), mode=lax.GatherScatterMode.PROMISE_IN_BOUNDS)


def _sc_fused(tab, idx_flat, wpk):
    @functools.partial(
        pl.kernel, mesh=plsc.VectorSubcoreMesh(**_MESH),
        compiler_params=_CP,
        out_type=jax.ShapeDtypeStruct((B,), jnp.float32),
        scratch_types=[
            pltpu.VMEM((IPS,), jnp.int32),
            pltpu.VMEM((IPS,), jnp.int32),
            pltpu.VMEM((IPS, RW), jnp.float32),
            pltpu.VMEM((IPS, RW), jnp.float32),
            pltpu.VMEM((WPK,), jnp.float32),
            pltpu.VMEM((SB,), jnp.float32),
            pltpu.SemaphoreType.DMA,
            pltpu.SemaphoreType.DMA,
        ],
    )
    def k(tab_hbm, idx_hbm, wpk_hbm, out_hbm,
          idx0, idx1, rows0, rows1, wv, outv, sem0, sem1):
        wid = lax.axis_index("s") * NC + lax.axis_index("c")
        pltpu.sync_copy(wpk_hbm, wv)
        ibase = wid * SPW * F
        obase = wid * SPW
        idx_bufs, row_bufs, sems = (idx0, idx1), (rows0, rows1), (sem0, sem1)
        iota16 = jnp.arange(16, dtype=jnp.int32)

        def fire(s, buf):
            off = ibase + s * IPS
            pltpu.sync_copy(idx_hbm.at[pl.ds(off, IPS)], idx_bufs[buf])
            for j in range(GA):
                pltpu.async_copy(
                    tab_hbm.at[idx_bufs[buf].at[pl.ds(j * IPG, IPG)]],
                    row_bufs[buf].at[pl.ds(j * IPG, IPG)], sems[buf])

        def drain(buf):
            pltpu.make_async_copy(tab_hbm.at[pl.ds(0, IPS)],
                                  row_bufs[buf], sems[buf]).wait()

        def compute(s, buf):
            rows_v = row_bufs[buf]
            for gg in range(GPS // 2):
                gA, gB = 2 * gg, 2 * gg + 1
                rA = (gA * 16 + iota16) * F
                rB = (gB * 16 + iota16) * F
                zero = jnp.zeros((16,), jnp.float32)
                init = (tuple([zero] * H1), tuple([zero] * H1), zero, zero)

                def fbody(f, carry):
                    accA, accB, bsA, bsB = carry
                    ra, rb = rA + f, rB + f
                    naA, naB = list(accA), list(accB)
                    for d in range(D):
                        cd = jnp.full((16,), d, jnp.int32)
                        xa = plsc.load_gather(rows_v, [ra, cd])
                        xb = plsc.load_gather(rows_v, [rb, cd])
                        wrow = wv[pl.ds(OFF_W1 + f * (D * 16) + d * 16, 16)]
                        for j in range(H1):
                            wb = _bcast(wrow, j)
                            naA[j] = naA[j] + xa * wb
                            naB[j] = naB[j] + xb * wb
                    c4 = jnp.full((16,), D, jnp.int32)
                    bsA = bsA + plsc.load_gather(rows_v, [ra, c4])
                    bsB = bsB + plsc.load_gather(rows_v, [rb, c4])
                    return tuple(naA), tuple(naB), bsA, bsB

                accA, accB, bsA, bsB = lax.fori_loop(0, F, fbody, init)
                for g, acc, bs in ((gA, accA, bsA), (gB, accB, bsB)):
                    h1 = [jnp.maximum(acc[j] + wv[pl.ds(OFF_B1 + j * 16, 16)],
                                      0.0)
                          for j in range(H1)]
                    pred = bs + wv[pl.ds(OFF_B3, 16)]
                    for m in range(H2):
                        t = wv[pl.ds(OFF_B2 + m * 16, 16)]
                        for j in range(H1):
                            t = t + h1[j] * wv[pl.ds(OFF_W2 + (m * H1 + j) * 16,
                                                     16)]
                        h2m = jnp.maximum(t, 0.0)
                        pred = pred + h2m * wv[pl.ds(OFF_W3 + m * 16, 16)]
                    outv[pl.ds(g * 16, 16)] = pred
            pltpu.sync_copy(outv, out_hbm.at[pl.ds(obase + s * SB, SB)])

        fire(0, 0)

        def step_pair(i, carry):
            sA = 2 * i
            fire(sA + 1, 1)
            drain(0)
            compute(sA, 0)

            @pl.when(sA + 2 < NST)
            def _():
                fire(sA + 2, 0)

            drain(1)
            compute(sA + 1, 1)
            return carry

        lax.fori_loop(0, NST // 2, step_pair, 0)

    return k(tab, idx_flat, wpk)


def _pack_weights(W1, b1, W2, b2, W3, b3):
    return jnp.concatenate([
        W1.T.reshape(-1),
        jnp.repeat(b1, 16),
        jnp.repeat(W2.reshape(-1), 16),
        jnp.repeat(b2, 16),
        jnp.repeat(W3[0], 16),
        jnp.repeat(b3, 16),
    ])


def kernel(fids_batch, emb_w, emb_b, W1, b1, W2, b2, W3, b3):
    tab = _sc_build_tab(emb_w.T, emb_b)
    idx_flat = fids_batch.reshape(NI)
    wpk = _pack_weights(W1, b1, W2, b2, W3, b3)
    return _sc_fused(tab, idx_flat, wpk)


# speedup vs baseline: 1.3485x; 1.3485x over previous
import functools

import jax
import jax.numpy as jnp
from jax import lax
from jax.experimental import pallas as pl
from jax.experimental.pallas import tpu as pltpu
from jax.experimental.pallas import tpu_sc as plsc

B, F, V, D = 16384, 66, 100000, 4
H1, H2 = 16, 8
RW = 8
NI = B * F
NC, NS = 2, 16
NW = NC * NS
IPG = 128

SB = 64
SPW = B // NW
NST = SPW // SB
GPS = SB // 16
IPS = SB * F
GA = IPS // IPG

OFF_W1 = 0
OFF_B1 = OFF_W1 + F * D * H1
OFF_W2 = OFF_B1 + H1 * 16
OFF_B2 = OFF_W2 + H2 * H1 * 16
OFF_W3 = OFF_B2 + H2 * 16
OFF_B3 = OFF_W3 + H2 * 16
WPK = OFF_B3 + 16

TB_CH = 2000
TB_NCH = V // TB_CH
TB_IT = -(-TB_NCH // NW)

_MESH = dict(core_axis_name="c", subcore_axis_name="s")
_CP = pltpu.CompilerParams(use_tc_tiling_on_sc=False, needs_layout_passes=False)


def _sc_build_tab(emb_wT, emb_b):
    @functools.partial(
        pl.kernel, mesh=plsc.VectorSubcoreMesh(**_MESH),
        compiler_params=_CP,
        out_type=jax.ShapeDtypeStruct((V, RW), jnp.float32),
        scratch_types=[
            pltpu.VMEM((D, TB_CH), jnp.float32),
            pltpu.VMEM((TB_CH,), jnp.float32),
            pltpu.VMEM((TB_CH, RW), jnp.float32),
        ],
    )
    def k(wT_hbm, b_hbm, tab_hbm, w_v, b_v, rows_v):
        wid = lax.axis_index("s") * NC + lax.axis_index("c")
        for it in range(TB_IT):
            c = wid + NW * it

            @pl.when(c < TB_NCH)
            def _():
                r0 = c * TB_CH
                pltpu.sync_copy(wT_hbm.at[:, pl.ds(r0, TB_CH)], w_v)
                pltpu.sync_copy(b_hbm.at[pl.ds(r0, TB_CH)], b_v)

                def body(g, carry):
                    rows = jnp.arange(16, dtype=jnp.int32) + g * 16
                    for d in range(D):
                        plsc.store_scatter(
                            rows_v,
                            [rows, jnp.full((16,), d, jnp.int32)],
                            w_v.at[d][pl.ds(g * 16, 16)])
                    plsc.store_scatter(
                        rows_v,
                        [rows, jnp.full((16,), D, jnp.int32)],
                        b_v[pl.ds(g * 16, 16)])
                    return carry

                lax.fori_loop(0, TB_CH // 16, body, 0)
                pltpu.sync_copy(rows_v, tab_hbm.at[pl.ds(r0, TB_CH)])

    return k(emb_wT, emb_b)


def _bcast(vec16, j):
    return lax.gather(
        vec16, jnp.full((16, 1), j, jnp.int32),
        lax.GatherDimensionNumbers(offset_dims=(), collapsed_slice_dims=(0,),
                                   start_index_map=(0,)),
        (1,), mode=lax.GatherScatterMode.PROMISE_IN_BOUNDS)


def _sc_fused(tab, idx_flat, wpk):
    @functools.partial(
        pl.kernel, mesh=plsc.VectorSubcoreMesh(**_MESH),
        compiler_params=_CP,
        out_type=jax.ShapeDtypeStruct((B,), jnp.float32),
        scratch_types=[
            pltpu.VMEM((IPS,), jnp.int32),
            pltpu.VMEM((IPS,), jnp.int32),
            pltpu.VMEM((IPS, RW), jnp.float32),
            pltpu.VMEM((IPS, RW), jnp.float32),
            pltpu.VMEM((WPK,), jnp.float32),
            pltpu.VMEM((SB,), jnp.float32),
            pltpu.VMEM((2 * H1 * 16,), jnp.float32),
            pltpu.VMEM((32,), jnp.float32),
            pltpu.SemaphoreType.DMA,
            pltpu.SemaphoreType.DMA,
        ],
    )
    def k(tab_hbm, idx_hbm, wpk_hbm, out_hbm,
          idx0, idx1, rows0, rows1, wv, outv, h1v, bsv, sem0, sem1):
        wid = lax.axis_index("s") * NC + lax.axis_index("c")
        pltpu.sync_copy(wpk_hbm, wv)
        ibase = wid * SPW * F
        obase = wid * SPW
        idx_bufs, row_bufs, sems = (idx0, idx1), (rows0, rows1), (sem0, sem1)
        iota16 = jnp.arange(16, dtype=jnp.int32)

        def fire(s, buf):
            off = ibase + s * IPS
            pltpu.sync_copy(idx_hbm.at[pl.ds(off, IPS)], idx_bufs[buf])
            for j in range(GA):
                pltpu.async_copy(
                    tab_hbm.at[idx_bufs[buf].at[pl.ds(j * IPG, IPG)]],
                    row_bufs[buf].at[pl.ds(j * IPG, IPG)], sems[buf])

        def drain(buf):
            pltpu.make_async_copy(tab_hbm.at[pl.ds(0, IPS)],
                                  row_bufs[buf], sems[buf]).wait()

        def compute(s, buf):
            rows_v = row_bufs[buf]

            def ggbody(gg, carry):
                rA = (gg * 32 + iota16) * F
                rB = (gg * 32 + 16 + iota16) * F
                zero = jnp.zeros((16,), jnp.float32)
                for jh in range(2):
                    init = tuple([zero] * 16) + ((zero, zero) if jh == 0
                                                 else ())

                    def fbody(f, carry, jh=jh):
                        accs = list(carry[:16])
                        ra, rb = rA + f, rB + f
                        for d in range(D):
                            cd = jnp.full((16,), d, jnp.int32)
                            xa = plsc.load_gather(rows_v, [ra, cd])
                            xb = plsc.load_gather(rows_v, [rb, cd])
                            wrow = wv[pl.ds(OFF_W1 + f * (D * 16) + d * 16,
                                            16)]
                            for jj in range(8):
                                wb = _bcast(wrow, jh * 8 + jj)
                                accs[jj] = accs[jj] + xa * wb
                                accs[8 + jj] = accs[8 + jj] + xb * wb
                        if jh == 0:
                            c4 = jnp.full((16,), D, jnp.int32)
                            bsA = carry[16] + plsc.load_gather(rows_v,
                                                               [ra, c4])
                            bsB = carry[17] + plsc.load_gather(rows_v,
                                                               [rb, c4])
                            return tuple(accs) + (bsA, bsB)
                        return tuple(accs)

                    res = lax.fori_loop(0, F, fbody, init)
                    for jj in range(8):
                        j = jh * 8 + jj
                        b1j = wv[pl.ds(OFF_B1 + j * 16, 16)]
                        h1v[pl.ds(j * 16, 16)] = jnp.maximum(
                            res[jj] + b1j, 0.0)
                        h1v[pl.ds(256 + j * 16, 16)] = jnp.maximum(
                            res[8 + jj] + b1j, 0.0)
                    if jh == 0:
                        bsv[pl.ds(0, 16)] = res[16]
                        bsv[pl.ds(16, 16)] = res[17]
                for g2 in range(2):
                    pred = bsv[pl.ds(g2 * 16, 16)] + wv[pl.ds(OFF_B3, 16)]
                    for m in range(H2):
                        t = wv[pl.ds(OFF_B2 + m * 16, 16)]
                        for j in range(H1):
                            t = t + (h1v[pl.ds(g2 * 256 + j * 16, 16)]
                                     * wv[pl.ds(OFF_W2 + (m * H1 + j) * 16,
                                                16)])
                        pred = pred + (jnp.maximum(t, 0.0)
                                       * wv[pl.ds(OFF_W3 + m * 16, 16)])
                    outv[pl.ds(gg * 32 + g2 * 16, 16)] = pred
                return carry

            lax.fori_loop(0, GPS // 2, ggbody, 0)
            pltpu.sync_copy(outv, out_hbm.at[pl.ds(obase + s * SB, SB)])

        fire(0, 0)

        def step_pair(i, carry):
            sA = 2 * i
            fire(sA + 1, 1)
            drain(0)
            compute(sA, 0)

            @pl.when(sA + 2 < NST)
            def _():
                fire(sA + 2, 0)

            drain(1)
            compute(sA + 1, 1)
            return carry

        lax.fori_loop(0, NST // 2, step_pair, 0)

    return k(tab, idx_flat, wpk)


def _pack_weights(W1, b1, W2, b2, W3, b3):
    return jnp.concatenate([
        W1.T.reshape(-1),
        jnp.repeat(b1, 16),
        jnp.repeat(W2.reshape(-1), 16),
        jnp.repeat(b2, 16),
        jnp.repeat(W3[0], 16),
        jnp.repeat(b3, 16),
    ])


def kernel(fids_batch, emb_w, emb_b, W1, b1, W2, b2, W3, b3):
    tab = _sc_build_tab(emb_w.T, emb_b)
    idx_flat = fids_batch.reshape(NI)
    wpk = _pack_weights(W1, b1, W2, b2, W3, b3)
    return _sc_fused(tab, idx_flat, wpk)
